# baseline (speedup 1.0000x reference)
import functools

import jax
import jax.numpy as jnp
from jax import lax
from jax.experimental import pallas as pl
from jax.experimental.pallas import tpu as pltpu
from jax.experimental.pallas import tpu_sc as plsc

EMBED_DIM = 32
PACK = 4
CHUNK = 128
NBUF = 4
GROUPS = CHUNK // 16


@functools.cache
def _make_kernel(n_flat: int, vocab: int):
    info = plsc.get_sparse_core_info()
    num_workers = info.num_cores * info.num_subcores
    b_per_w = n_flat // num_workers
    steps = b_per_w // CHUNK
    mesh = plsc.VectorSubcoreMesh(core_axis_name="c", subcore_axis_name="s")

    @functools.partial(
        pl.kernel,
        mesh=mesh,
        out_type=jax.ShapeDtypeStruct((n_flat * EMBED_DIM,), jnp.float32),
        scratch_types=[
            pltpu.VMEM((NBUF, CHUNK), jnp.int32),
            pltpu.VMEM((NBUF, CHUNK), jnp.int32),
            *[pltpu.VMEM((CHUNK, 128), jnp.float32) for _ in range(NBUF)],
            *[pltpu.VMEM((CHUNK * EMBED_DIM,), jnp.float32) for _ in range(NBUF)],
            pltpu.SemaphoreType.DMA,
            pltpu.SemaphoreType.DMA,
            pltpu.SemaphoreType.DMA,
        ],
        compiler_params=pltpu.CompilerParams(needs_layout_passes=False),
    )
    def emb_kernel(idx_hbm, table_hbm, out_hbm, ibuf, dbuf, *rest):
        gbufs = rest[:NBUF]
        obufs = rest[NBUF : 2 * NBUF]
        isem, gsem, osem = rest[2 * NBUF : 2 * NBUF + 3]
        wid = lax.axis_index("s") * info.num_cores + lax.axis_index("c")
        base = wid * b_per_w
        my_idx = idx_hbm.at[wid]

        iota = lax.iota(jnp.int32, 16)

        def fetch_idx(j, islot):
            pltpu.async_copy(my_idx.at[j], ibuf.at[islot], isem)

        def wait_idx(islot):
            pltpu.make_async_copy(my_idx.at[0], ibuf.at[islot], isem).wait()

        def div_and_gather(slot):
            for g in range(GROUPS):
                iv = ibuf[slot, pl.ds(g * 16, 16)]
                dbuf[slot, pl.ds(g * 16, 16)] = lax.shift_right_logical(iv, 2)
            pltpu.async_copy(table_hbm.at[dbuf.at[slot]], gbufs[slot], gsem)

        def wait_gather(slot):
            pltpu.make_async_copy(
                table_hbm.at[dbuf.at[slot]], gbufs[slot], gsem
            ).wait()

        iota_hi = iota + 16
        gather_dnums = lax.GatherDimensionNumbers(
            offset_dims=(), collapsed_slice_dims=(0,), start_index_map=(0,)
        )

        def bcast_lane(vec16, k):
            starts = jnp.full((16, 1), k, jnp.int32)
            return lax.gather(
                vec16,
                starts,
                gather_dnums,
                (1,),
                mode=lax.GatherScatterMode.PROMISE_IN_BOUNDS,
            )

        def extract(slot):
            for g in range(GROUPS):
                iv = ibuf[slot, pl.ds(g * 16, 16)]
                colbase = lax.shift_left(jnp.bitwise_and(iv, PACK - 1), 5)
                for k in range(16):
                    i = g * 16 + k
                    row = jnp.full((16,), i, jnp.int32)
                    cb = bcast_lane(colbase, k)
                    lo = plsc.load_gather(gbufs[slot], [row, cb + iota])
                    hi = plsc.load_gather(gbufs[slot], [row, cb + iota_hi])
                    obufs[slot][pl.ds(i * EMBED_DIM, 16)] = lo
                    obufs[slot][pl.ds(i * EMBED_DIM + 16, 16)] = hi

        def fire_out(j, slot):
            pltpu.async_copy(
                obufs[slot],
                out_hbm.at[pl.ds((base + j * CHUNK) * EMBED_DIM, CHUNK * EMBED_DIM)],
                osem,
            )

        def drain_out(slot):
            pltpu.make_async_copy(
                obufs[slot],
                out_hbm.at[pl.ds(0, CHUNK * EMBED_DIM)],
                osem,
            ).wait()

        for p in range(4):
            fetch_idx(p, p % NBUF)
        for p in range(3):
            wait_idx(p)
            div_and_gather(p)

        def outer(jo, _):
            for b in range(NBUF):
                j = jo * NBUF + b
                b3 = (b + 3) % NBUF

                @pl.when(j + 3 < steps)
                def _gather():
                    wait_idx(b3)
                    div_and_gather(b3)

                wait_gather(b)

                @pl.when(j >= NBUF)
                def _drain():
                    drain_out(b)

                extract(b)
                fire_out(j, b)

                @pl.when(j + 4 < steps)
                def _fetch():
                    fetch_idx(j + 4, b)

            return 0

        lax.fori_loop(0, steps // NBUF, outer, 0)
        for b in range(NBUF):
            drain_out(b)

    return emb_kernel, num_workers, steps


def kernel(x, weight):
    batch, hist = x.shape
    vocab, dim = weight.shape
    n_flat = batch * hist
    emb, num_workers, steps = _make_kernel(n_flat, vocab)
    idx = x.reshape(num_workers, steps, CHUNK)
    table = weight.reshape(vocab // PACK, dim * PACK)
    out = emb(idx, table)
    return out.reshape(batch, hist, dim)


# speedup vs baseline: 1.1115x; 1.1115x over previous
import functools

import jax
import jax.numpy as jnp
from jax import lax
from jax.experimental import pallas as pl
from jax.experimental.pallas import tpu as pltpu
from jax.experimental.pallas import tpu_sc as plsc

EMBED_DIM = 32
PACK = 4
CHUNK = 128
NBUF = 4
GROUPS = CHUNK // 16


@functools.cache
def _make_kernel(n_flat: int, vocab: int):
    info = plsc.get_sparse_core_info()
    num_workers = info.num_cores * info.num_subcores
    b_per_w = n_flat // num_workers
    steps = b_per_w // CHUNK
    mesh = plsc.VectorSubcoreMesh(core_axis_name="c", subcore_axis_name="s")

    @functools.partial(
        pl.kernel,
        mesh=mesh,
        out_type=jax.ShapeDtypeStruct((n_flat * EMBED_DIM,), jnp.float32),
        scratch_types=[
            pltpu.VMEM((NBUF, CHUNK), jnp.int32),
            pltpu.VMEM((NBUF, CHUNK), jnp.int32),
            *[pltpu.VMEM((CHUNK, 128), jnp.float32) for _ in range(NBUF)],
            *[pltpu.VMEM((CHUNK * EMBED_DIM,), jnp.float32) for _ in range(NBUF)],
            pltpu.SemaphoreType.DMA,
            pltpu.SemaphoreType.DMA,
            pltpu.SemaphoreType.DMA,
        ],
        compiler_params=pltpu.CompilerParams(needs_layout_passes=False),
    )
    def emb_kernel(idx_hbm, table_hbm, out_hbm, ibuf, dbuf, *rest):
        gbufs = rest[:NBUF]
        obufs = rest[NBUF : 2 * NBUF]
        isem, gsem, osem = rest[2 * NBUF : 2 * NBUF + 3]
        wid = lax.axis_index("s") * info.num_cores + lax.axis_index("c")
        base = wid * b_per_w
        my_idx = idx_hbm.at[wid]

        iota = lax.iota(jnp.int32, 16)

        def fetch_idx(j, islot):
            pltpu.async_copy(my_idx.at[j], ibuf.at[islot], isem)

        def wait_idx(islot):
            pltpu.make_async_copy(my_idx.at[0], ibuf.at[islot], isem).wait()

        def div_and_gather(slot):
            for g in range(GROUPS):
                iv = ibuf[slot, pl.ds(g * 16, 16)]
                dbuf[slot, pl.ds(g * 16, 16)] = lax.shift_right_logical(iv, 2)
            pltpu.async_copy(table_hbm.at[dbuf.at[slot]], gbufs[slot], gsem)

        def wait_gather(slot):
            pltpu.make_async_copy(
                table_hbm.at[dbuf.at[slot]], gbufs[slot], gsem
            ).wait()

        iota_hi = iota + 16
        gather_dnums = lax.GatherDimensionNumbers(
            offset_dims=(), collapsed_slice_dims=(0,), start_index_map=(0,)
        )

        def bcast_lane(vec16, k):
            starts = jnp.full((16, 1), k, jnp.int32)
            return lax.gather(
                vec16,
                starts,
                gather_dnums,
                (1,),
                mode=lax.GatherScatterMode.PROMISE_IN_BOUNDS,
            )

        def extract(slot):
            for g in range(GROUPS):
                iv = ibuf[slot, pl.ds(g * 16, 16)]
                colbase = lax.shift_left(jnp.bitwise_and(iv, PACK - 1), 5)
                for k in range(16):
                    i = g * 16 + k
                    row = jnp.full((16,), i, jnp.int32)
                    cb = bcast_lane(colbase, k)
                    lo = plsc.load_gather(gbufs[slot], [row, cb + iota])
                    hi = plsc.load_gather(gbufs[slot], [row, cb + iota_hi])
                    obufs[slot][pl.ds(i * EMBED_DIM, 16)] = lo
                    obufs[slot][pl.ds(i * EMBED_DIM + 16, 16)] = hi

        def fire_out(j, slot):
            pltpu.async_copy(
                obufs[slot],
                out_hbm.at[pl.ds((base + j * CHUNK) * EMBED_DIM, CHUNK * EMBED_DIM)],
                osem,
            )

        def drain_out(slot):
            pltpu.make_async_copy(
                obufs[slot],
                out_hbm.at[pl.ds(0, CHUNK * EMBED_DIM)],
                osem,
            ).wait()

        for p in range(4):
            fetch_idx(p, p % NBUF)
        wait_idx(0)
        div_and_gather(0)
        wait_idx(1)
        div_and_gather(1)

        def outer(jo, _):
            for b in range(NBUF):
                j = jo * NBUF + b
                b2 = (b + 2) % NBUF

                @pl.when(j + 2 < steps)
                def _gather():
                    wait_idx(b2)
                    div_and_gather(b2)

                wait_gather(b)

                @pl.when(j >= NBUF)
                def _drain():
                    drain_out(b)

                extract(b)
                fire_out(j, b)

                @pl.when(j + 4 < steps)
                def _fetch():
                    fetch_idx(j + 4, b)

            return 0

        lax.fori_loop(0, steps // NBUF, outer, 0)
        for b in range(NBUF):
            drain_out(b)

    return emb_kernel, num_workers, steps


def kernel(x, weight):
    batch, hist = x.shape
    vocab, dim = weight.shape
    n_flat = batch * hist
    emb, num_workers, steps = _make_kernel(n_flat, vocab)
    idx = x.reshape(num_workers, steps, CHUNK)
    table = weight.reshape(vocab // PACK, dim * PACK)
    out = emb(idx, table)
    return out.reshape(batch, hist, dim)
